# Initial kernel scaffold; baseline (speedup 1.0000x reference)
import jax
import jax.numpy as jnp
from jax.experimental import pallas as pl


def kernel(x, embedding):
    raise NotImplementedError("write your pallas kernel here")


# speedup vs baseline: 3.2038x; 3.2038x over previous
import functools

import jax
import jax.numpy as jnp
from jax import lax
from jax.experimental import pallas as pl
from jax.experimental.pallas import tpu as pltpu

D = 8
K = 512
DIM = 512
SUBDIM = DIM // D
TN = 1000


def _tc_body(x_ref, emb_ref, embt_ref, sn_ref, en_ref, idxt_ref, recon_ref,
             loss_ref):
    i = pl.program_id(0)
    acc = jnp.float32(0.0)
    for d in range(D):
        xd = x_ref[:, d * SUBDIM:(d + 1) * SUBDIM]
        et = embt_ref[d]
        e = emb_ref[d]
        c = jnp.dot(xd, et, preferred_element_type=jnp.float32)
        sn = sn_ref[:, d:d + 1]
        en = en_ref[d:d + 1, :]
        dist = sn + en - 2.0 * c
        mn = jnp.min(dist, axis=1)
        iota_k = lax.broadcasted_iota(jnp.int32, (TN, K), 1)
        a = jnp.min(jnp.where(dist == mn[:, None], iota_k, jnp.int32(K)),
                    axis=1)
        idxt_ref[0, d:d + 1, :] = a[None, :]
        oh = (iota_k == a[:, None]).astype(jnp.float32)
        q = jnp.dot(oh, e, preferred_element_type=jnp.float32)
        recon_ref[:, d * SUBDIM:(d + 1) * SUBDIM] = q
        acc = acc + jnp.sum(mn)

    @pl.when(i == 0)
    def _():
        loss_ref[:, :] = jnp.zeros((1, 1), jnp.float32)

    loss_ref[:, :] += jnp.reshape(acc, (1, 1))


def kernel(x, embedding):
    B, T, _ = x.shape
    flat = x.reshape(-1, DIM)
    N = flat.shape[0]
    G = N // TN
    embt = jnp.transpose(embedding, (0, 2, 1))
    sn = jnp.sum(flat.reshape(N, D, SUBDIM) ** 2, axis=-1)
    en = jnp.sum(embedding ** 2, axis=-1)

    idxt, recon, loss = pl.pallas_call(
        _tc_body,
        grid=(G,),
        in_specs=[
            pl.BlockSpec((TN, DIM), lambda i: (i, 0)),
            pl.BlockSpec((D, K, SUBDIM), lambda i: (0, 0, 0)),
            pl.BlockSpec((D, SUBDIM, K), lambda i: (0, 0, 0)),
            pl.BlockSpec((TN, D), lambda i: (i, 0)),
            pl.BlockSpec((D, K), lambda i: (0, 0)),
        ],
        out_specs=[
            pl.BlockSpec((1, D, TN), lambda i: (i, 0, 0)),
            pl.BlockSpec((TN, DIM), lambda i: (i, 0)),
            pl.BlockSpec((1, 1), lambda i: (0, 0)),
        ],
        out_shape=[
            jax.ShapeDtypeStruct((G, D, TN), jnp.int32),
            jax.ShapeDtypeStruct((N, DIM), jnp.float32),
            jax.ShapeDtypeStruct((1, 1), jnp.float32),
        ],
    )(flat, embedding, embt, sn, en)

    indices = idxt.transpose(0, 2, 1).reshape(B, T, D)
    vq_loss = (loss[0, 0] * (1.25 / (N * DIM))).astype(jnp.float32)
    return recon.reshape(B, T, DIM), indices, vq_loss
